# baseline (speedup 1.0000x reference)
import functools

import jax
import jax.numpy as jnp
from jax import lax
from jax.experimental import pallas as pl
from jax.experimental.pallas import tpu as pltpu
from jax.experimental.pallas import tpu_sc as plsc

NC = 2
NS = 16
NW = NC * NS
NBUF = 4
LANES = 16


@functools.partial(jax.jit, static_argnames=("bw", "hist", "d"))
def _sc_gather(idx, weight, *, bw, hist, d):
  outer = hist // NBUF
  et_n, es_n = d // 8, 8
  bl_n = bw // LANES
  mesh = plsc.VectorSubcoreMesh(
      core_axis_name="c", subcore_axis_name="s",
      num_cores=NC, num_subcores=NS)

  @functools.partial(
      pl.kernel,
      out_type=jax.ShapeDtypeStruct((hist, et_n, NW, es_n, bw), jnp.float32),
      mesh=mesh,
      scratch_types=[
          pltpu.VMEM((bw * hist,), jnp.int32),
          pltpu.VMEM((hist, bw), jnp.int32),
          pltpu.VMEM((NBUF, bw, d), jnp.float32),
          pltpu.VMEM((NBUF, et_n, es_n, bw), jnp.float32),
          pltpu.SemaphoreType.DMA((NBUF,)),
          pltpu.SemaphoreType.DMA((NBUF,)),
      ],
      compiler_params=pltpu.CompilerParams(
          use_tc_tiling_on_sc=False, needs_layout_passes=False),
  )
  def body(idx_hbm, table_hbm, out_hbm, idx_v, idxt_v, rows_v, trans_v,
           sem_in, sem_out):
    wid = lax.axis_index("s") * NC + lax.axis_index("c")
    i16 = lax.iota(jnp.int32, LANES)
    pltpu.sync_copy(idx_hbm.at[pl.ds(wid * bw * hist, bw * hist)], idx_v)

    @pl.loop(0, hist)
    def _(h):
      for b0 in range(bl_n):
        vec = plsc.load_gather(idx_v, [(i16 + b0 * LANES) * hist + h])
        idxt_v[h, pl.ds(b0 * LANES, LANES)] = vec

    for b in range(NBUF):
      pltpu.async_copy(
          table_hbm.at[idxt_v.at[b]], rows_v.at[b], sem_in.at[b])

    @pl.loop(0, outer)
    def _(g):
      for b in range(NBUF):
        h = g * NBUF + b
        pltpu.make_async_copy(
            table_hbm.at[idxt_v.at[h]], rows_v.at[b], sem_in.at[b]).wait()
        @pl.when(g > 0)
        def _():
          pltpu.make_async_copy(
              trans_v.at[b], out_hbm.at[h, :, wid], sem_out.at[b]).wait()

        @pl.loop(0, et_n)
        def _(et):
          for es in range(es_n):
            e = et * es_n + es
            for b0 in range(bl_n):
              vec = plsc.load_gather(
                  rows_v.at[b], [i16 + b0 * LANES, jnp.full((LANES,), e, jnp.int32)])
              trans_v[b, et, es, pl.ds(b0 * LANES, LANES)] = vec

        pltpu.async_copy(trans_v.at[b], out_hbm.at[h, :, wid], sem_out.at[b])

        @pl.when(g < outer - 1)
        def _():
          pltpu.async_copy(
              table_hbm.at[idxt_v.at[h + NBUF]], rows_v.at[b], sem_in.at[b])

    for b in range(NBUF):
      h = (outer - 1) * NBUF + b
      pltpu.make_async_copy(
          trans_v.at[b], out_hbm.at[h, :, wid], sem_out.at[b]).wait()

  return body(idx, weight)


def kernel(inputs, weight):
  batch, hist = inputs.shape
  d = weight.shape[1]
  assert batch % (NW * LANES) == 0 and hist % NBUF == 0 and d % 8 == 0
  bw = batch // NW
  idx = jax.lax.optimization_barrier(
      inputs.astype(jnp.int32).reshape(batch * hist))
  wt = jax.lax.optimization_barrier(weight.reshape(-1)).reshape(
      weight.shape[0], d)
  out5 = _sc_gather(idx, wt, bw=bw, hist=hist, d=d)
  out = out5.transpose(2, 4, 0, 1, 3).reshape(batch, hist, d)
  return out


# speedup vs baseline: 1.5019x; 1.5019x over previous
import functools

import jax
import jax.numpy as jnp
from jax import lax
from jax.experimental import pallas as pl
from jax.experimental.pallas import tpu as pltpu
from jax.experimental.pallas import tpu_sc as plsc

NC = 2
NS = 16
NW = NC * NS
NBUF = 4


@functools.partial(jax.jit, static_argnames=("bw", "hist", "d"))
def _sc_gather(idx, weight, *, bw, hist, d):
  outer = bw // NBUF
  mesh = plsc.VectorSubcoreMesh(
      core_axis_name="c", subcore_axis_name="s",
      num_cores=NC, num_subcores=NS)

  @functools.partial(
      pl.kernel,
      out_type=jax.ShapeDtypeStruct((NW * bw, hist, d), jnp.float32),
      mesh=mesh,
      scratch_types=[
          pltpu.VMEM((bw * hist,), jnp.int32),
          pltpu.VMEM((NBUF, hist, d), jnp.float32),
          pltpu.SemaphoreType.DMA((NBUF,)),
          pltpu.SemaphoreType.DMA((NBUF,)),
      ],
      compiler_params=pltpu.CompilerParams(use_tc_tiling_on_sc=False),
  )
  def body(idx_hbm, table_hbm, out_hbm, idx_v, rows_v, sem_in, sem_out):
    wid = lax.axis_index("s") * NC + lax.axis_index("c")
    pltpu.sync_copy(idx_hbm.at[pl.ds(wid * bw * hist, bw * hist)], idx_v)
    for b in range(NBUF):
      pltpu.async_copy(
          table_hbm.at[idx_v.at[pl.ds(b * hist, hist)]],
          rows_v.at[b], sem_in.at[b])

    @pl.loop(0, outer)
    def _(g):
      for b in range(NBUF):
        r = g * NBUF + b
        pltpu.make_async_copy(
            table_hbm.at[idx_v.at[pl.ds(r * hist, hist)]],
            rows_v.at[b], sem_in.at[b]).wait()
        pltpu.async_copy(rows_v.at[b], out_hbm.at[wid * bw + r], sem_out.at[b])

        @pl.when(g < outer - 1)
        def _():
          pltpu.make_async_copy(
              rows_v.at[b], out_hbm.at[wid * bw + r], sem_out.at[b]).wait()
          pltpu.async_copy(
              table_hbm.at[idx_v.at[pl.ds((r + NBUF) * hist, hist)]],
              rows_v.at[b], sem_in.at[b])

    for b in range(NBUF):
      r = (outer - 1) * NBUF + b
      pltpu.make_async_copy(
          rows_v.at[b], out_hbm.at[wid * bw + r], sem_out.at[b]).wait()

  return body(idx, weight)


def kernel(inputs, weight):
  batch, hist = inputs.shape
  d = weight.shape[1]
  assert batch % (NW * NBUF) == 0
  bw = batch // NW
  idx = jax.lax.optimization_barrier(
      inputs.astype(jnp.int32).reshape(batch * hist))
  wt = jax.lax.optimization_barrier(weight.reshape(-1)).reshape(
      weight.shape[0], d)
  return _sc_gather(idx, wt, bw=bw, hist=hist, d=d)


# speedup vs baseline: 1.5071x; 1.0035x over previous
import functools

import jax
import jax.numpy as jnp
from jax import lax
from jax.experimental import pallas as pl
from jax.experimental.pallas import tpu as pltpu
from jax.experimental.pallas import tpu_sc as plsc

NC = 2
NS = 16
NW = NC * NS
NBUF = 4


@functools.partial(jax.jit, static_argnames=("bw", "hist", "d"))
def _sc_gather(idx, weight, *, bw, hist, d):
  outer = bw // NBUF
  mesh = plsc.VectorSubcoreMesh(
      core_axis_name="c", subcore_axis_name="s",
      num_cores=NC, num_subcores=NS)

  @functools.partial(
      pl.kernel,
      out_type=jax.ShapeDtypeStruct((NW * bw, hist, d), jnp.float32),
      mesh=mesh,
      scratch_types=[
          pltpu.VMEM((bw * hist,), jnp.int32),
          pltpu.VMEM((NBUF, hist, d), jnp.float32),
          pltpu.SemaphoreType.DMA((NBUF,)),
          pltpu.SemaphoreType.DMA((NBUF,)),
      ],
      compiler_params=pltpu.CompilerParams(use_tc_tiling_on_sc=False),
  )
  def body(idx_hbm, table_hbm, out_hbm, idx_v, rows_v, sem_in, sem_out):
    wid = lax.axis_index("s") * NC + lax.axis_index("c")
    pltpu.sync_copy(idx_hbm.at[pl.ds(wid * bw * hist, bw * hist)], idx_v)
    for b in range(NBUF):
      pltpu.async_copy(
          table_hbm.at[idx_v.at[pl.ds(b * hist, hist)]],
          rows_v.at[b], sem_in.at[b])

    @pl.loop(0, outer)
    def _(g):
      for b in range(NBUF):
        r = g * NBUF + b
        pltpu.make_async_copy(
            table_hbm.at[idx_v.at[pl.ds(r * hist, hist)]],
            rows_v.at[b], sem_in.at[b]).wait()
        pltpu.async_copy(rows_v.at[b], out_hbm.at[wid * bw + r], sem_out.at[b])

        @pl.when(g < outer - 1)
        def _():
          pltpu.make_async_copy(
              rows_v.at[b], out_hbm.at[wid * bw + r], sem_out.at[b]).wait()
          pltpu.async_copy(
              table_hbm.at[idx_v.at[pl.ds((r + NBUF) * hist, hist)]],
              rows_v.at[b], sem_in.at[b])

    for b in range(NBUF):
      r = (outer - 1) * NBUF + b
      pltpu.make_async_copy(
          rows_v.at[b], out_hbm.at[wid * bw + r], sem_out.at[b]).wait()

  return body(idx, weight)


def kernel(inputs, weight):
  batch, hist = inputs.shape
  d = weight.shape[1]
  assert batch % (NW * NBUF) == 0
  bw = batch // NW
  idx = jax.lax.optimization_barrier(
      inputs.astype(jnp.int32).reshape(batch * hist))
  from jax.experimental.layout import Format, Layout
  sharding = jax.sharding.SingleDeviceSharding(jax.devices()[0])
  wt = jax.device_put(
      weight,
      Format(Layout(major_to_minor=(0, 1), tiling=((8,),)), sharding))
  return _sc_gather(idx, wt, bw=bw, hist=hist, d=d)
